# Initial kernel scaffold; baseline (speedup 1.0000x reference)
import jax
import jax.numpy as jnp
from jax.experimental import pallas as pl


def kernel(H, knn_idx, W, b):
    raise NotImplementedError("write your pallas kernel here")


# speedup vs baseline: 36.4136x; 36.4136x over previous
import functools

import jax
import jax.numpy as jnp
from jax import lax
from jax.experimental import pallas as pl
from jax.experimental.pallas import tpu as pltpu
from jax.experimental.pallas import tpu_sc as plsc

_NC = 2
_NS = 16
_NW = _NC * _NS


def _sc_gather_sum(table, idx3, K):
    R, HD = table.shape
    NW, NCH, CBK = idx3.shape
    CB = CBK // K
    RPT = R // NW
    NH = HD // 16

    mesh = plsc.VectorSubcoreMesh(core_axis_name="c", subcore_axis_name="s")

    @functools.partial(
        pl.kernel,
        out_type=jax.ShapeDtypeStruct((R, HD), jnp.float32),
        mesh=mesh,
        compiler_params=pltpu.CompilerParams(use_tc_tiling_on_sc=False),
        scratch_types=[
            pltpu.VMEM((NCH, CBK), jnp.int32),
            pltpu.VMEM((CBK, HD), jnp.float32),
            pltpu.VMEM((CBK, HD), jnp.float32),
            pltpu.VMEM((CB, HD), jnp.float32),
            pltpu.VMEM((CB, HD), jnp.float32),
            pltpu.SemaphoreType.DMA,
            pltpu.SemaphoreType.DMA,
            pltpu.SemaphoreType.DMA,
            pltpu.SemaphoreType.DMA,
        ],
    )
    def k(tab_hbm, idx_hbm, out_hbm, idx_v, g0, g1, o0, o1, sg0, sg1, so0, so1):
        wid = lax.axis_index("s") * _NC + lax.axis_index("c")
        base_row = wid * RPT
        pltpu.sync_copy(idx_hbm.at[wid], idx_v)
        gbufs = (g0, g1)
        obufs = (o0, o1)
        gsems = (sg0, sg1)
        osems = (so0, so1)
        pltpu.make_async_copy(tab_hbm.at[idx_v.at[0]], g0, sg0).start()
        pltpu.make_async_copy(tab_hbm.at[idx_v.at[1]], g1, sg1).start()

        def pair(i, carry):
            for b2 in range(2):
                c = i * 2 + b2
                gb, ob = gbufs[b2], obufs[b2]
                gs, os_ = gsems[b2], osems[b2]
                pltpu.make_async_copy(tab_hbm.at[idx_v.at[c]], gb, gs).wait()
                @pl.when(c >= 2)
                def _():
                    pltpu.make_async_copy(
                        ob, out_hbm.at[pl.ds(base_row, CB)], os_).wait()

                def row(r, rc):
                    rb = r * K
                    for h in range(NH):
                        s = pl.ds(h * 16, 16)
                        a = gb[rb, s]
                        for kk in range(1, K):
                            a = a + gb[rb + kk, s]
                        ob[r, s] = a
                    return rc

                lax.fori_loop(0, CB, row, 0)
                @pl.when(c + 2 < NCH)
                def _():
                    pltpu.make_async_copy(
                        tab_hbm.at[idx_v.at[c + 2]], gb, gs).start()
                pltpu.make_async_copy(
                    ob, out_hbm.at[pl.ds(base_row + c * CB, CB)], os_).start()
            return carry

        lax.fori_loop(0, NCH // 2, pair, 0)
        pltpu.make_async_copy(o0, out_hbm.at[pl.ds(base_row, CB)], so0).wait()
        pltpu.make_async_copy(o1, out_hbm.at[pl.ds(base_row, CB)], so1).wait()

    return k(table, idx3)


def _tc_combine(Hf, G, W, b2, K):
    N, DHD = Hf.shape
    D, _, HD = G.shape
    BN = 1000
    scale = 1.0 / K

    def body(h_ref, g_ref, w_ref, b_ref, o_ref):
        w = w_ref[...]
        bb = b_ref[...]
        for d in range(D):
            g = g_ref[d] * scale
            m = lax.dot_general(g, w, (((1,), (1,)), ((), ())),
                                preferred_element_type=jnp.float32)
            o_ref[:, d * HD:(d + 1) * HD] = h_ref[:, d * HD:(d + 1) * HD] + m + bb

    return pl.pallas_call(
        body,
        grid=(N // BN,),
        in_specs=[
            pl.BlockSpec((BN, DHD), lambda i: (i, 0)),
            pl.BlockSpec((D, BN, HD), lambda i: (0, i, 0)),
            pl.BlockSpec((HD, HD), lambda i: (0, 0)),
            pl.BlockSpec((1, HD), lambda i: (0, 0)),
        ],
        out_specs=pl.BlockSpec((BN, DHD), lambda i: (i, 0)),
        out_shape=jax.ShapeDtypeStruct((N, DHD), jnp.float32),
    )(Hf, G, W, b2)


def kernel(H, knn_idx, W, b):
    N, D, HD = H.shape
    K = knn_idx.shape[-1]
    CB = 5
    H2 = H.reshape(N * D, HD)
    offs = jnp.arange(D, dtype=jnp.int32)[:, None, None]
    idx3 = (knn_idx * D + offs).reshape(_NW, -1, CB * K)
    G = _sc_gather_sum(H2, idx3, K)
    out = _tc_combine(H.reshape(N, D * HD), G.reshape(D, N, HD),
                      W, b.reshape(1, HD), K)
    return out.reshape(N, D, HD)
